# baseline (speedup 1.0000x reference)
import functools

import jax
import jax.numpy as jnp
import numpy as np
from jax import lax
from jax.experimental import pallas as pl
from jax.experimental.pallas import tpu as pltpu
from jax.experimental.pallas import tpu_sc as plsc

_N_SPINS = 20
_BATCH = 16384
_HID = 32
_PACK = 4
_ROWS_ALL = _BATCH // _PACK
_KDIM = _N_SPINS * _PACK
_LANES = _HID * _PACK

_LN2 = 0.6931471805599453
_LOG2E = 1.4426950408889634
_INV_2PI = 0.15915494309189535
_TWO_PI_HI = np.float32(6.2831855)
_TWO_PI_LO = np.float32(2 * np.pi - np.float64(np.float32(6.2831855)))
_HALF_PI = 1.5707963267948966
_SQRT2 = 1.4142135623730951

_P2 = (1.00000005, 0.6931472, 0.24022212, 0.05550341, 0.00967077,
       0.00133953)
_SIN = (9.99999862e-01, -1.66666077e-01, 8.33273244e-03, -1.98166923e-04,
        2.70832613e-06, -2.06959702e-08)
_COS = (9.99999974e-01, -4.99999851e-01, 4.16664624e-02, -1.38877318e-03,
        2.47690534e-05, -2.70754507e-07, 1.72437522e-09)
_LOG = (2.00860633e-08, 9.99999939e-01, -5.00007396e-01, 3.33348268e-01,
        -2.49588182e-01, 1.99077502e-01, -1.73609514e-01, 1.61652754e-01,
        -9.71980421e-02)
_ATN = (0.9999984, -0.3332385, 0.19861805, -0.13427489, 0.08302168,
        -0.03645597, 0.00773056)


def _poly(x, coefs):
    terms = list(coefs)
    p = x
    while len(terms) > 1:
        nxt = [terms[i] + terms[i + 1] * p for i in range(0, len(terms) - 1, 2)]
        if len(terms) % 2:
            nxt.append(terms[-1])
        terms = nxt
        p = p * p
    return terms[0]


_BLK = 2048
_CHUNK = 128
_N_CHUNKS = _BLK // _CHUNK


def _mlp_body(x_ref, w1rh_ref, w1rl_ref, w1ih_ref, w1il_ref, b1r_ref,
              b1i_ref, wri_ref, wir_ref, b2i_ref, imag_ref,
              zr_s, zi_s, yr_s, yi_s):
    dn = (((1,), (0,)), ((), ()))
    hi = lax.Precision.HIGHEST

    xb = x_ref[...].astype(jnp.bfloat16)
    zr_s[...] = (
        lax.dot_general(xb, w1rh_ref[...], dn,
                        preferred_element_type=jnp.float32)
        + lax.dot_general(xb, w1rl_ref[...], dn,
                          preferred_element_type=jnp.float32))
    zi_s[...] = (
        lax.dot_general(xb, w1ih_ref[...], dn,
                        preferred_element_type=jnp.float32)
        + lax.dot_general(xb, w1il_ref[...], dn,
                          preferred_element_type=jnp.float32))

    def chunk(j, _):
        sl = pl.ds(j * _CHUNK, _CHUNK)
        zr = zr_s[sl, :] + b1r_ref[...]
        zi = zi_s[sl, :] + b1i_ref[...]
        a = jnp.abs(zr)
        b = jnp.where(zr < 0.0, -zi, zi)

        t = jnp.exp(-2.0 * a)

        xb = 2.0 * b
        k = jnp.round(xb * _INV_2PI)
        rb = (xb - k * _TWO_PI_HI) - k * _TWO_PI_LO
        rb2 = rb * rb
        sinv = rb * _poly(rb2, _SIN)
        cosv = _poly(rb2, _COS)

        u = t * cosv
        v = -t * sinv

        w = 1.0 + 2.0 * u + t * t
        iw = lax.bitcast_convert_type(w, jnp.int32)
        e = ((iw >> 23) - 127).astype(jnp.float32)
        m = lax.bitcast_convert_type((iw & 0x007FFFFF) | 0x3F800000,
                                     jnp.float32)
        big = m > _SQRT2
        m = jnp.where(big, 0.5 * m, m)
        e = jnp.where(big, e + 1.0, e)
        lnw = e * _LN2 + _poly(m - 1.0, _LOG)
        yr = a + 0.5 * lnw - _LN2

        d = 1.0 + u
        av = jnp.abs(v)
        num = jnp.minimum(av, d)
        den = jnp.maximum(av, d)
        zq = num / den
        at = zq * _poly(zq * zq, _ATN)
        at = jnp.where(av > d, _HALF_PI - at, at)
        yi = b + jnp.where(v < 0.0, -at, at)

        yr_s[sl, :] = yr.astype(jnp.bfloat16)
        yi_s[sl, :] = yi.astype(jnp.bfloat16)
        return 0

    lax.fori_loop(0, _N_CHUNKS, chunk, 0)

    imag_ref[...] = (
        lax.dot_general(yr_s[...], wri_ref[...], dn,
                        preferred_element_type=jnp.float32)
        + lax.dot_general(yi_s[...], wir_ref[...], dn,
                          preferred_element_type=jnp.float32)
        + b2i_ref[0])


def _mlp_call(x2, w1rh, w1rl, w1ih, w1il, b1r, b1i, wri, wir, b2i):
    n_blocks = _ROWS_ALL // _BLK
    full = lambda shape: pl.BlockSpec(shape, lambda i: (0,) * len(shape))
    return pl.pallas_call(
        _mlp_body,
        grid=(n_blocks,),
        in_specs=[
            pl.BlockSpec((_BLK, _KDIM), lambda i: (i, 0)),
            full((_KDIM, _LANES)),
            full((_KDIM, _LANES)),
            full((_KDIM, _LANES)),
            full((_KDIM, _LANES)),
            full((_LANES,)),
            full((_LANES,)),
            full((_LANES, _PACK)),
            full((_LANES, _PACK)),
            full((1,)),
        ],
        out_specs=pl.BlockSpec((_BLK, _PACK), lambda i: (i, 0)),
        out_shape=jax.ShapeDtypeStruct((_ROWS_ALL, _PACK), jnp.float32),
        scratch_shapes=[
            pltpu.VMEM((_BLK, _LANES), jnp.float32),
            pltpu.VMEM((_BLK, _LANES), jnp.float32),
            pltpu.VMEM((_BLK, _LANES), jnp.bfloat16),
            pltpu.VMEM((_BLK, _LANES), jnp.bfloat16),
        ],
    )(x2, w1rh, w1rl, w1ih, w1il, b1r, b1i, wri, wir, b2i)


_NW = 32
_IROWS = 128
_ICOLS = 128
_ROWS_PER_W = _IROWS // _NW


def _gather_body(table_hbm, x_hbm, out_hbm, xcols, idx_rows, vals_v, sem):
    wid = lax.axis_index("s") * 2 + lax.axis_index("c")
    base = wid * _ELEM_PER_W
    col_copies = [
        pltpu.async_copy(x_hbm.at[pl.ds(s * _BATCH + base, _ELEM_PER_W)],
                         xcols.at[pl.ds(s * _ELEM_PER_W, _ELEM_PER_W)], sem)
        for s in range(_N_SPINS)
    ]
    for c in col_copies:
        c.wait()
    for g in range(_ELEM_PER_W // 16):
        acc = jnp.zeros((16,), jnp.int32)
        for s in range(_N_SPINS):
            xvals = xcols[pl.ds(s * _ELEM_PER_W + g * 16, 16)]
            bit = jnp.where(xvals > 0.0, 1, 0).astype(jnp.int32)
            acc = (acc << 1) | bit
        idx_rows[g // 8, pl.ds((g % 8) * 16, 16)] = acc
    copies = [
        pltpu.async_copy(table_hbm.at[idx_rows.at[j]], vals_v.at[j], sem)
        for j in range(_ROWS_PER_W)
    ]
    for c in copies:
        c.wait()
    pltpu.sync_copy(vals_v, out_hbm.at[pl.ds(wid * _ROWS_PER_W, _ROWS_PER_W)])


_ELEM_PER_W = _BATCH // _NW


@functools.cache
def _gather_call():
    return pl.kernel(
        _gather_body,
        out_type=jax.ShapeDtypeStruct((_IROWS, _ICOLS), jnp.float32),
        mesh=plsc.VectorSubcoreMesh(core_axis_name="c", subcore_axis_name="s"),
        scratch_types=[
            pltpu.VMEM((_N_SPINS * _ELEM_PER_W,), jnp.float32),
            pltpu.VMEM((_ROWS_PER_W, _ICOLS), jnp.int32),
            pltpu.VMEM((_ROWS_PER_W, _ICOLS), jnp.float32),
            pltpu.SemaphoreType.DMA,
        ],
    )


def kernel(x, exact_log_amps, W1r, W1i, b1r, b1i, W2r, W2i, b2r, b2i):
    eye = jnp.eye(_PACK, dtype=jnp.float32)
    w1rb = jnp.kron(eye, W1r)
    w1ib = jnp.kron(eye, W1i)
    w1rh = w1rb.astype(jnp.bfloat16)
    w1rl = (w1rb - w1rh.astype(jnp.float32)).astype(jnp.bfloat16)
    w1ih = w1ib.astype(jnp.bfloat16)
    w1il = (w1ib - w1ih.astype(jnp.float32)).astype(jnp.bfloat16)
    b1rb = jnp.tile(b1r, _PACK)
    b1ib = jnp.tile(b1i, _PACK)
    wri = jnp.kron(eye, W2i).astype(jnp.bfloat16)
    wir = jnp.kron(eye, W2r).astype(jnp.bfloat16)
    x2 = x.reshape(_ROWS_ALL, _KDIM)
    xt = x.T.reshape(_N_SPINS * _BATCH)
    real = _gather_call()(exact_log_amps, xt)
    imag = _mlp_call(x2, w1rh, w1rl, w1ih, w1il, b1rb, b1ib, wri, wir, b2i)
    return lax.complex(real.reshape(_BATCH), imag.reshape(_BATCH))


# speedup vs baseline: 1.0001x; 1.0001x over previous
import functools

import jax
import jax.numpy as jnp
import numpy as np
from jax import lax
from jax.experimental import pallas as pl
from jax.experimental.pallas import tpu as pltpu
from jax.experimental.pallas import tpu_sc as plsc

_N_SPINS = 20
_BATCH = 16384
_HID = 32
_PACK = 4
_ROWS_ALL = _BATCH // _PACK
_KDIM = _N_SPINS * _PACK
_LANES = _HID * _PACK

_LN2 = 0.6931471805599453
_LOG2E = 1.4426950408889634
_INV_2PI = 0.15915494309189535
_TWO_PI_HI = np.float32(6.2831855)
_TWO_PI_LO = np.float32(2 * np.pi - np.float64(np.float32(6.2831855)))
_HALF_PI = 1.5707963267948966
_SQRT2 = 1.4142135623730951

_P2 = (1.00000005, 0.6931472, 0.24022212, 0.05550341, 0.00967077,
       0.00133953)
_SIN = (9.99999862e-01, -1.66666077e-01, 8.33273244e-03, -1.98166923e-04,
        2.70832613e-06, -2.06959702e-08)
_COS = (9.99999974e-01, -4.99999851e-01, 4.16664624e-02, -1.38877318e-03,
        2.47690534e-05, -2.70754507e-07, 1.72437522e-09)
_LOG = (2.00860633e-08, 9.99999939e-01, -5.00007396e-01, 3.33348268e-01,
        -2.49588182e-01, 1.99077502e-01, -1.73609514e-01, 1.61652754e-01,
        -9.71980421e-02)
_ATN = (0.9999984, -0.3332385, 0.19861805, -0.13427489, 0.08302168,
        -0.03645597, 0.00773056)


def _poly(x, coefs):
    terms = list(coefs)
    p = x
    while len(terms) > 1:
        nxt = [terms[i] + terms[i + 1] * p for i in range(0, len(terms) - 1, 2)]
        if len(terms) % 2:
            nxt.append(terms[-1])
        terms = nxt
        p = p * p
    return terms[0]


_BLK = 2048
_CHUNK = 128
_N_CHUNKS = _BLK // _CHUNK


def _mlp_body(x_ref, w1rh_ref, w1rl_ref, w1ih_ref, w1il_ref, b1r_ref,
              b1i_ref, wri_ref, wir_ref, b2i_ref, imag_ref,
              zr_s, zi_s, yr_s, yi_s):
    dn = (((1,), (0,)), ((), ()))
    hi = lax.Precision.HIGHEST

    xb = x_ref[...].astype(jnp.bfloat16)
    zr_s[...] = (
        lax.dot_general(xb, w1rh_ref[...], dn,
                        preferred_element_type=jnp.float32)
        + lax.dot_general(xb, w1rl_ref[...], dn,
                          preferred_element_type=jnp.float32))
    zi_s[...] = (
        lax.dot_general(xb, w1ih_ref[...], dn,
                        preferred_element_type=jnp.float32)
        + lax.dot_general(xb, w1il_ref[...], dn,
                          preferred_element_type=jnp.float32))

    def chunk(j, _):
        sl = pl.ds(j * _CHUNK, _CHUNK)
        zr = zr_s[sl, :] + b1r_ref[...]
        zi = zi_s[sl, :] + b1i_ref[...]
        a = jnp.abs(zr)
        b = jnp.where(zr < 0.0, -zi, zi)

        t = jnp.exp(-2.0 * a)

        xb = 2.0 * b
        k = jnp.round(xb * _INV_2PI)
        rb = (xb - k * _TWO_PI_HI) - k * _TWO_PI_LO
        rb2 = rb * rb
        sinv = rb * _poly(rb2, _SIN)
        cosv = _poly(rb2, _COS)

        u = t * cosv
        v = -t * sinv

        w = 1.0 + 2.0 * u + t * t
        iw = lax.bitcast_convert_type(w, jnp.int32)
        e = ((iw >> 23) - 127).astype(jnp.float32)
        m = lax.bitcast_convert_type((iw & 0x007FFFFF) | 0x3F800000,
                                     jnp.float32)
        big = m > _SQRT2
        m = jnp.where(big, 0.5 * m, m)
        e = jnp.where(big, e + 1.0, e)
        lnw = e * _LN2 + _poly(m - 1.0, _LOG)
        yr = a + 0.5 * lnw - _LN2

        d = 1.0 + u
        av = jnp.abs(v)
        num = jnp.minimum(av, d)
        den = jnp.maximum(av, d)
        zq = num / den
        at = zq * _poly(zq * zq, _ATN)
        at = jnp.where(av > d, _HALF_PI - at, at)
        yi = b + jnp.where(v < 0.0, -at, at)

        yr_s[sl, :] = yr
        yi_s[sl, :] = yi
        return 0

    lax.fori_loop(0, _N_CHUNKS, chunk, 0)

    imag_ref[...] = (
        lax.dot_general(yr_s[...], wri_ref[...], dn,
                        preferred_element_type=jnp.float32)
        + lax.dot_general(yi_s[...], wir_ref[...], dn,
                          preferred_element_type=jnp.float32)
        + b2i_ref[0])


def _mlp_call(x2, w1rh, w1rl, w1ih, w1il, b1r, b1i, wri, wir, b2i):
    n_blocks = _ROWS_ALL // _BLK
    full = lambda shape: pl.BlockSpec(shape, lambda i: (0,) * len(shape))
    return pl.pallas_call(
        _mlp_body,
        grid=(n_blocks,),
        in_specs=[
            pl.BlockSpec((_BLK, _KDIM), lambda i: (i, 0)),
            full((_KDIM, _LANES)),
            full((_KDIM, _LANES)),
            full((_KDIM, _LANES)),
            full((_KDIM, _LANES)),
            full((_LANES,)),
            full((_LANES,)),
            full((_LANES, _PACK)),
            full((_LANES, _PACK)),
            full((1,)),
        ],
        out_specs=pl.BlockSpec((_BLK, _PACK), lambda i: (i, 0)),
        out_shape=jax.ShapeDtypeStruct((_ROWS_ALL, _PACK), jnp.float32),
        scratch_shapes=[
            pltpu.VMEM((_BLK, _LANES), jnp.float32),
            pltpu.VMEM((_BLK, _LANES), jnp.float32),
            pltpu.VMEM((_BLK, _LANES), jnp.float32),
            pltpu.VMEM((_BLK, _LANES), jnp.float32),
        ],
    )(x2, w1rh, w1rl, w1ih, w1il, b1r, b1i, wri, wir, b2i)


_NW = 32
_IROWS = 128
_ICOLS = 128
_ROWS_PER_W = _IROWS // _NW


def _gather_body(table_hbm, x_hbm, out_hbm, xcols, idx_rows, vals_v, sem):
    wid = lax.axis_index("s") * 2 + lax.axis_index("c")
    base = wid * _ELEM_PER_W
    col_copies = [
        pltpu.async_copy(x_hbm.at[pl.ds(s * _BATCH + base, _ELEM_PER_W)],
                         xcols.at[pl.ds(s * _ELEM_PER_W, _ELEM_PER_W)], sem)
        for s in range(_N_SPINS)
    ]
    for c in col_copies:
        c.wait()
    for g in range(_ELEM_PER_W // 16):
        acc = jnp.zeros((16,), jnp.int32)
        for s in range(_N_SPINS):
            xvals = xcols[pl.ds(s * _ELEM_PER_W + g * 16, 16)]
            bit = jnp.where(xvals > 0.0, 1, 0).astype(jnp.int32)
            acc = (acc << 1) | bit
        idx_rows[g // 8, pl.ds((g % 8) * 16, 16)] = acc
    copies = [
        pltpu.async_copy(table_hbm.at[idx_rows.at[j]], vals_v.at[j], sem)
        for j in range(_ROWS_PER_W)
    ]
    for c in copies:
        c.wait()
    pltpu.sync_copy(vals_v, out_hbm.at[pl.ds(wid * _ROWS_PER_W, _ROWS_PER_W)])


_ELEM_PER_W = _BATCH // _NW


@functools.cache
def _gather_call():
    return pl.kernel(
        _gather_body,
        out_type=jax.ShapeDtypeStruct((_IROWS, _ICOLS), jnp.float32),
        mesh=plsc.VectorSubcoreMesh(core_axis_name="c", subcore_axis_name="s"),
        scratch_types=[
            pltpu.VMEM((_N_SPINS * _ELEM_PER_W,), jnp.float32),
            pltpu.VMEM((_ROWS_PER_W, _ICOLS), jnp.int32),
            pltpu.VMEM((_ROWS_PER_W, _ICOLS), jnp.float32),
            pltpu.SemaphoreType.DMA,
        ],
    )


def kernel(x, exact_log_amps, W1r, W1i, b1r, b1i, W2r, W2i, b2r, b2i):
    eye = jnp.eye(_PACK, dtype=jnp.float32)
    w1rb = jnp.kron(eye, W1r)
    w1ib = jnp.kron(eye, W1i)
    w1rh = w1rb.astype(jnp.bfloat16)
    w1rl = (w1rb - w1rh.astype(jnp.float32)).astype(jnp.bfloat16)
    w1ih = w1ib.astype(jnp.bfloat16)
    w1il = (w1ib - w1ih.astype(jnp.float32)).astype(jnp.bfloat16)
    b1rb = jnp.tile(b1r, _PACK)
    b1ib = jnp.tile(b1i, _PACK)
    wri = jnp.kron(eye, W2i)
    wir = jnp.kron(eye, W2r)
    x2 = x.reshape(_ROWS_ALL, _KDIM)
    xt = x.T.reshape(_N_SPINS * _BATCH)
    real = _gather_call()(exact_log_amps, xt)
    imag = _mlp_call(x2, w1rh, w1rl, w1ih, w1il, b1rb, b1ib, wri, wir, b2i)
    return lax.complex(real.reshape(_BATCH), imag.reshape(_BATCH))


# speedup vs baseline: 1.0054x; 1.0053x over previous
import functools

import jax
import jax.numpy as jnp
import numpy as np
from jax import lax
from jax.experimental import pallas as pl
from jax.experimental.pallas import tpu as pltpu
from jax.experimental.pallas import tpu_sc as plsc

_N_SPINS = 20
_BATCH = 16384
_HID = 32
_PACK = 4
_ROWS_ALL = _BATCH // _PACK
_KDIM = _N_SPINS * _PACK
_LANES = _HID * _PACK

_LN2 = 0.6931471805599453
_LOG2E = 1.4426950408889634
_INV_2PI = 0.15915494309189535
_TWO_PI_HI = np.float32(6.2831855)
_TWO_PI_LO = np.float32(2 * np.pi - np.float64(np.float32(6.2831855)))
_HALF_PI = 1.5707963267948966
_SQRT2 = 1.4142135623730951

_P2 = (1.00000005, 0.6931472, 0.24022212, 0.05550341, 0.00967077,
       0.00133953)
_SIN = (9.99999862e-01, -1.66666077e-01, 8.33273244e-03, -1.98166923e-04,
        2.70832613e-06, -2.06959702e-08)
_COS = (9.99999974e-01, -4.99999851e-01, 4.16664624e-02, -1.38877318e-03,
        2.47690534e-05, -2.70754507e-07, 1.72437522e-09)
_LOG = (2.00860633e-08, 9.99999939e-01, -5.00007396e-01, 3.33348268e-01,
        -2.49588182e-01, 1.99077502e-01, -1.73609514e-01, 1.61652754e-01,
        -9.71980421e-02)
_ATN = (0.9999984, -0.3332385, 0.19861805, -0.13427489, 0.08302168,
        -0.03645597, 0.00773056)


def _poly(x, coefs):
    terms = list(coefs)
    p = x
    while len(terms) > 1:
        nxt = [terms[i] + terms[i + 1] * p for i in range(0, len(terms) - 1, 2)]
        if len(terms) % 2:
            nxt.append(terms[-1])
        terms = nxt
        p = p * p
    return terms[0]


_BLK = 2048
_CHUNK = 256
_N_CHUNKS = _BLK // _CHUNK


def _mlp_body(x_ref, w1rh_ref, w1rl_ref, w1ih_ref, w1il_ref, b1r_ref,
              b1i_ref, wri_ref, wir_ref, b2i_ref, imag_ref,
              zr_s, zi_s, yr_s, yi_s):
    dn = (((1,), (0,)), ((), ()))

    xb = x_ref[...].astype(jnp.bfloat16)
    zr_s[...] = (
        lax.dot_general(xb, w1rh_ref[...], dn,
                        preferred_element_type=jnp.float32)
        + lax.dot_general(xb, w1rl_ref[...], dn,
                          preferred_element_type=jnp.float32))
    zi_s[...] = (
        lax.dot_general(xb, w1ih_ref[...], dn,
                        preferred_element_type=jnp.float32)
        + lax.dot_general(xb, w1il_ref[...], dn,
                          preferred_element_type=jnp.float32))

    def chunk(j, _):
        sl = pl.ds(j * _CHUNK, _CHUNK)
        zr = zr_s[sl, :] + b1r_ref[...]
        zi = zi_s[sl, :] + b1i_ref[...]
        a = jnp.abs(zr)
        b = jnp.where(zr < 0.0, -zi, zi)

        t = jnp.exp(-2.0 * a)

        xb = 2.0 * b
        k = jnp.round(xb * _INV_2PI)
        rb = (xb - k * _TWO_PI_HI) - k * _TWO_PI_LO
        rb2 = rb * rb
        sinv = rb * _poly(rb2, _SIN)
        cosv = _poly(rb2, _COS)

        u = t * cosv
        v = -t * sinv

        w = 1.0 + 2.0 * u + t * t
        iw = lax.bitcast_convert_type(w, jnp.int32)
        e = ((iw >> 23) - 127).astype(jnp.float32)
        m = lax.bitcast_convert_type((iw & 0x007FFFFF) | 0x3F800000,
                                     jnp.float32)
        big = m > _SQRT2
        m = jnp.where(big, 0.5 * m, m)
        e = jnp.where(big, e + 1.0, e)
        lnw = e * _LN2 + _poly(m - 1.0, _LOG)
        yr = a + 0.5 * lnw - _LN2

        d = 1.0 + u
        av = jnp.abs(v)
        num = jnp.minimum(av, d)
        den = jnp.maximum(av, d)
        zq = num / den
        at = zq * _poly(zq * zq, _ATN)
        at = jnp.where(av > d, _HALF_PI - at, at)
        yi = b + jnp.where(v < 0.0, -at, at)

        yr_s[sl, :] = yr
        yi_s[sl, :] = yi
        return 0

    lax.fori_loop(0, _N_CHUNKS, chunk, 0)

    imag_ref[...] = (
        lax.dot_general(yr_s[...], wri_ref[...], dn,
                        preferred_element_type=jnp.float32)
        + lax.dot_general(yi_s[...], wir_ref[...], dn,
                          preferred_element_type=jnp.float32)
        + b2i_ref[0])


def _mlp_call(x2, w1rh, w1rl, w1ih, w1il, b1r, b1i, wri, wir, b2i):
    n_blocks = _ROWS_ALL // _BLK
    full = lambda shape: pl.BlockSpec(shape, lambda i: (0,) * len(shape))
    return pl.pallas_call(
        _mlp_body,
        grid=(n_blocks,),
        in_specs=[
            pl.BlockSpec((_BLK, _KDIM), lambda i: (i, 0)),
            full((_KDIM, _LANES)),
            full((_KDIM, _LANES)),
            full((_KDIM, _LANES)),
            full((_KDIM, _LANES)),
            full((_LANES,)),
            full((_LANES,)),
            full((_LANES, _PACK)),
            full((_LANES, _PACK)),
            full((1,)),
        ],
        out_specs=pl.BlockSpec((_BLK, _PACK), lambda i: (i, 0)),
        out_shape=jax.ShapeDtypeStruct((_ROWS_ALL, _PACK), jnp.float32),
        scratch_shapes=[
            pltpu.VMEM((_BLK, _LANES), jnp.float32),
            pltpu.VMEM((_BLK, _LANES), jnp.float32),
            pltpu.VMEM((_BLK, _LANES), jnp.float32),
            pltpu.VMEM((_BLK, _LANES), jnp.float32),
        ],
    )(x2, w1rh, w1rl, w1ih, w1il, b1r, b1i, wri, wir, b2i)


_NW = 32
_IROWS = 128
_ICOLS = 128
_ROWS_PER_W = _IROWS // _NW


def _gather_body(table_hbm, x_hbm, out_hbm, xcols, idx_rows, vals_v, sem):
    wid = lax.axis_index("s") * 2 + lax.axis_index("c")
    base = wid * _ELEM_PER_W
    col_copies = [
        pltpu.async_copy(x_hbm.at[pl.ds(s * _BATCH + base, _ELEM_PER_W)],
                         xcols.at[pl.ds(s * _ELEM_PER_W, _ELEM_PER_W)], sem)
        for s in range(_N_SPINS)
    ]
    for c in col_copies:
        c.wait()
    for g in range(_ELEM_PER_W // 16):
        acc = jnp.zeros((16,), jnp.int32)
        for s in range(_N_SPINS):
            xvals = xcols[pl.ds(s * _ELEM_PER_W + g * 16, 16)]
            bit = jnp.where(xvals > 0.0, 1, 0).astype(jnp.int32)
            acc = (acc << 1) | bit
        idx_rows[g // 8, pl.ds((g % 8) * 16, 16)] = acc
    copies = [
        pltpu.async_copy(table_hbm.at[idx_rows.at[j]], vals_v.at[j], sem)
        for j in range(_ROWS_PER_W)
    ]
    for c in copies:
        c.wait()
    pltpu.sync_copy(vals_v, out_hbm.at[pl.ds(wid * _ROWS_PER_W, _ROWS_PER_W)])


_ELEM_PER_W = _BATCH // _NW


@functools.cache
def _gather_call():
    return pl.kernel(
        _gather_body,
        out_type=jax.ShapeDtypeStruct((_IROWS, _ICOLS), jnp.float32),
        mesh=plsc.VectorSubcoreMesh(core_axis_name="c", subcore_axis_name="s"),
        scratch_types=[
            pltpu.VMEM((_N_SPINS * _ELEM_PER_W,), jnp.float32),
            pltpu.VMEM((_ROWS_PER_W, _ICOLS), jnp.int32),
            pltpu.VMEM((_ROWS_PER_W, _ICOLS), jnp.float32),
            pltpu.SemaphoreType.DMA,
        ],
    )


def kernel(x, exact_log_amps, W1r, W1i, b1r, b1i, W2r, W2i, b2r, b2i):
    eye = jnp.eye(_PACK, dtype=jnp.float32)
    w1rb = jnp.kron(eye, W1r)
    w1ib = jnp.kron(eye, W1i)
    w1rh = w1rb.astype(jnp.bfloat16)
    w1rl = (w1rb - w1rh.astype(jnp.float32)).astype(jnp.bfloat16)
    w1ih = w1ib.astype(jnp.bfloat16)
    w1il = (w1ib - w1ih.astype(jnp.float32)).astype(jnp.bfloat16)
    b1rb = jnp.tile(b1r, _PACK)
    b1ib = jnp.tile(b1i, _PACK)
    wri = jnp.kron(eye, W2i)
    wir = jnp.kron(eye, W2r)
    x2 = x.reshape(_ROWS_ALL, _KDIM)
    xt = x.T.reshape(_N_SPINS * _BATCH)
    real = _gather_call()(exact_log_amps, xt)
    imag = _mlp_call(x2, w1rh, w1rl, w1ih, w1il, b1rb, b1ib, wri, wir, b2i)
    return lax.complex(real.reshape(_BATCH), imag.reshape(_BATCH))
